# baseline (speedup 1.0000x reference)
import functools

import jax
import jax.numpy as jnp
from jax import lax
from jax.experimental import pallas as pl
from jax.experimental.pallas import tpu as pltpu
from jax.experimental.pallas import tpu_sc as plsc

SUB_KEYS = 1024
TOPK = 16
HALF = 512
VDIM = 16
NTOK = 8192
KK = TOPK * TOPK

TOK_BLK = 256
GRID = NTOK // TOK_BLK

NC, NS = 2, 16
NW = NC * NS
TPW = NTOK // NW
CHUNK = 8
ROWS = CHUNK * KK
NCH = TPW // CHUNK
IDX_PER_DMA = 128
NDMA = ROWS // IDX_PER_DMA


def _top16_t(s, row):
    vals, idxs = [], []
    for _ in range(TOPK):
        m = jnp.max(s, axis=0, keepdims=True)
        im = jnp.min(jnp.where(s == m, row, SUB_KEYS), axis=0, keepdims=True)
        vals.append(m)
        idxs.append(im)
        s = jnp.where(row == im, -jnp.inf, s)
    return jnp.concatenate(vals, 0), jnp.concatenate(idxs, 0)


NVAL = SUB_KEYS * SUB_KEYS
VSTRIPE = NVAL // GRID


def _tc_body(q_ref, c1_ref, c2_ref, vt_ref, fi_ref, wc_ref, vlin_ref):
    vb = vt_ref[...]
    v3 = jnp.transpose(vb).reshape(VSTRIPE // 8, 8, VDIM)
    vlin_ref[...] = jnp.concatenate([v3[:, e, :] for e in range(8)], axis=-1)
    q = q_ref[...]
    dn = (((1,), (1,)), ((), ()))
    s1 = lax.dot_general(c1_ref[...], q[:, :HALF], dn,
                         preferred_element_type=jnp.float32)
    s2 = lax.dot_general(c2_ref[...], q[:, HALF:], dn,
                         preferred_element_type=jnp.float32)
    row = lax.broadcasted_iota(jnp.int32, (SUB_KEYS, TOK_BLK), 0)
    v1, i1 = _top16_t(s1, row)
    v2, i2 = _top16_t(s2, row)
    e1 = jnp.exp(v1 - v1[0:1])
    w1 = e1 / jnp.sum(e1, axis=0, keepdims=True)
    e2 = jnp.exp(v2 - v2[0:1])
    w2 = e2 / jnp.sum(e2, axis=0, keepdims=True)
    fi_t = (i1[:, None, :] * SUB_KEYS + i2[None, :, :]).reshape(KK, TOK_BLK)
    wc_t = (w1[:, None, :] * w2[None, :, :]).reshape(KK, TOK_BLK)
    fi4 = fi_t.T.reshape(TOK_BLK // 8, 8, 2, 128).transpose(0, 2, 1, 3)
    wc4 = wc_t.T.reshape(TOK_BLK // 8, 8, 2, 128).transpose(0, 2, 1, 3)
    fi_ref[...] = fi4
    wc_ref[...] = wc4


_tc_call = pl.pallas_call(
    _tc_body,
    grid=(GRID,),
    in_specs=[
        pl.BlockSpec((TOK_BLK, 2 * HALF), lambda i: (i, 0)),
        pl.BlockSpec((SUB_KEYS, HALF), lambda i: (0, 0)),
        pl.BlockSpec((SUB_KEYS, HALF), lambda i: (0, 0)),
        pl.BlockSpec((VDIM, VSTRIPE), lambda i: (0, i)),
    ],
    out_specs=[
        pl.BlockSpec((TOK_BLK // 8, 2, 8, 128), lambda i: (i, 0, 0, 0)),
        pl.BlockSpec((TOK_BLK // 8, 2, 8, 128), lambda i: (i, 0, 0, 0)),
        pl.BlockSpec((VSTRIPE // 8, 128), lambda i: (i, 0)),
    ],
    out_shape=[
        jax.ShapeDtypeStruct((NTOK // 8, 2, 8, 128), jnp.int32),
        jax.ShapeDtypeStruct((NTOK // 8, 2, 8, 128), jnp.float32),
        jax.ShapeDtypeStruct((NVAL // 8, 128), jnp.float32),
    ],
)


def _sc_body(fi_hbm, w_hbm, val_hbm, out_hbm,
             idx0, idx1, w0, w1, rows0, rows1, ob_v,
             isem0, isem1, gsem0, gsem1):
    wid = lax.axis_index("s") * NC + lax.axis_index("c")
    tok0 = wid * TPW
    idx_b = (idx0, idx1)
    w_b = (w0, w1)
    rows_b = (rows0, rows1)
    isem_b = (isem0, isem1)
    gsem_b = (gsem0, gsem1)

    def fetch_idx(c, b):
        @pl.when(c < NCH)
        def _():
            br = tok0 // CHUNK + c
            pltpu.make_async_copy(fi_hbm.at[br], idx_b[b], isem_b[b]).start()
            pltpu.make_async_copy(w_hbm.at[br], w_b[b], isem_b[b]).start()

    def fire_gathers(c, b):
        @pl.when(c < NCH)
        def _():
            pltpu.make_async_copy(
                fi_hbm.at[0], idx_b[b], isem_b[b]).wait()
            pltpu.make_async_copy(
                w_hbm.at[0], w_b[b], isem_b[b]).wait()
            for t in range(CHUNK):
                for h in range(2):
                    pltpu.make_async_copy(
                        val_hbm.at[idx_b[b].at[h, t]],
                        rows_b[b].at[pl.ds((t * 2 + h) * IDX_PER_DMA,
                                           IDX_PER_DMA)],
                        gsem_b[b]).start()

    def compute(c, b):
        rows_v = rows_b[b]
        w_v = w_b[b]
        for t in range(CHUNK):
            for h in range(2):
                pltpu.make_async_copy(
                    val_hbm.at[idx_b[b].at[h, t]],
                    rows_v.at[pl.ds((t * 2 + h) * IDX_PER_DMA, IDX_PER_DMA)],
                    gsem_b[b]).wait()
        for t in range(CHUNK):
            z = jnp.zeros((VDIM,), jnp.float32)
            accs = (z, z, z, z)
            for h in range(2):
                def row_body(r, a_in, _t=t, _h=h):
                    a = list(a_in)
                    base = r * 32
                    wa = w_v[_h, _t, pl.ds(base, 16)]
                    wb = w_v[_h, _t, pl.ds(base + 16, 16)]
                    rowbase = _t * KK + _h * IDX_PER_DMA + base
                    for u in range(16):
                        a[u % 4] = a[u % 4] + rows_v[rowbase + u, :] * wa[u]
                    for u in range(16):
                        a[u % 4] = (a[u % 4]
                                    + rows_v[rowbase + 16 + u, :] * wb[u])
                    return tuple(a)
                accs = lax.fori_loop(0, IDX_PER_DMA // 32, row_body, accs)
            a0, a1, a2, a3 = accs
            ob_v[t, :] = (a0 + a1) + (a2 + a3)
        pltpu.sync_copy(ob_v, out_hbm.at[pl.ds(tok0 + c * CHUNK, CHUNK)])

    fetch_idx(0, 0)
    fetch_idx(1, 1)
    fire_gathers(0, 0)

    def pair_body(p, carry):
        c0 = 2 * p
        for b in (0, 1):
            c = c0 + b
            fire_gathers(c + 1, 1 - b)
            compute(c, b)
            fetch_idx(c + 2, b)
        return carry

    lax.fori_loop(0, NCH // 2, pair_body, 0)


@functools.cache
def _get_sc_call():
    return pl.kernel(
        _sc_body,
        out_type=jax.ShapeDtypeStruct((NTOK, VDIM), jnp.float32),
        mesh=plsc.VectorSubcoreMesh(core_axis_name="c", subcore_axis_name="s",
                                    num_cores=NC, num_subcores=NS),
        compiler_params=pltpu.CompilerParams(use_tc_tiling_on_sc=False),
        scratch_types=[
            pltpu.VMEM((2, CHUNK, IDX_PER_DMA), jnp.int32),
            pltpu.VMEM((2, CHUNK, IDX_PER_DMA), jnp.int32),
            pltpu.VMEM((2, CHUNK, IDX_PER_DMA), jnp.float32),
            pltpu.VMEM((2, CHUNK, IDX_PER_DMA), jnp.float32),
            pltpu.VMEM((ROWS, VDIM), jnp.float32),
            pltpu.VMEM((ROWS, VDIM), jnp.float32),
            pltpu.VMEM((CHUNK, VDIM), jnp.float32),
            pltpu.SemaphoreType.DMA,
            pltpu.SemaphoreType.DMA,
            pltpu.SemaphoreType.DMA,
            pltpu.SemaphoreType.DMA,
        ],
    )


def kernel(query, codebook1, codebook2, values):
    b, t, d = query.shape
    q = query.reshape(b * t, d)
    fi, wc, vlin = _tc_call(q, codebook1, codebook2, values.T)
    out = _get_sc_call()(fi, wc, vlin.reshape(NVAL, VDIM))
    return out.reshape(b, t, VDIM)


# speedup vs baseline: 1.2110x; 1.2110x over previous
import functools

import jax
import jax.numpy as jnp
from jax import lax
from jax.experimental import pallas as pl
from jax.experimental.pallas import tpu as pltpu
from jax.experimental.pallas import tpu_sc as plsc

SUB_KEYS = 1024
TOPK = 16
HALF = 512
VDIM = 16
NTOK = 8192
KK = TOPK * TOPK

NSPLIT = 2
NTOK_S = NTOK // NSPLIT

TOK_BLK = 256
GRID = NTOK_S // TOK_BLK

NC, NS = 2, 16
NW = NC * NS
TPW = NTOK_S // NW
CHUNK = 8
ROWS = CHUNK * KK
NCH = TPW // CHUNK
IDX_PER_DMA = 128
NDMA = ROWS // IDX_PER_DMA


def _top16_t(s, row):
    vals, idxs = [], []
    for _ in range(TOPK):
        m = jnp.max(s, axis=0, keepdims=True)
        im = jnp.min(jnp.where(s == m, row, SUB_KEYS), axis=0, keepdims=True)
        vals.append(m)
        idxs.append(im)
        s = jnp.where(row == im, -jnp.inf, s)
    return jnp.concatenate(vals, 0), jnp.concatenate(idxs, 0)


def _tc_body(q_ref, c1_ref, c2_ref, fi_ref, wc_ref):
    q = q_ref[...]
    dn = (((1,), (1,)), ((), ()))
    s1 = lax.dot_general(c1_ref[...], q[:, :HALF], dn,
                         preferred_element_type=jnp.float32)
    s2 = lax.dot_general(c2_ref[...], q[:, HALF:], dn,
                         preferred_element_type=jnp.float32)
    row = lax.broadcasted_iota(jnp.int32, (SUB_KEYS, TOK_BLK), 0)
    v1, i1 = _top16_t(s1, row)
    v2, i2 = _top16_t(s2, row)
    e1 = jnp.exp(v1 - v1[0:1])
    w1 = e1 / jnp.sum(e1, axis=0, keepdims=True)
    e2 = jnp.exp(v2 - v2[0:1])
    w2 = e2 / jnp.sum(e2, axis=0, keepdims=True)
    fi_t = (i1[:, None, :] * SUB_KEYS + i2[None, :, :]).reshape(KK, TOK_BLK)
    wc_t = (w1[:, None, :] * w2[None, :, :]).reshape(KK, TOK_BLK)
    fi4 = fi_t.T.reshape(TOK_BLK // 8, 8, 2, 128).transpose(0, 2, 1, 3)
    wc4 = wc_t.T.reshape(TOK_BLK // 8, 8, 2, 128).transpose(0, 2, 1, 3)
    fi_ref[...] = fi4
    wc_ref[...] = wc4


_tc_call = pl.pallas_call(
    _tc_body,
    grid=(GRID,),
    in_specs=[
        pl.BlockSpec((TOK_BLK, 2 * HALF), lambda i: (i, 0)),
        pl.BlockSpec((SUB_KEYS, HALF), lambda i: (0, 0)),
        pl.BlockSpec((SUB_KEYS, HALF), lambda i: (0, 0)),
    ],
    out_specs=[
        pl.BlockSpec((TOK_BLK // 8, 2, 8, 128), lambda i: (i, 0, 0, 0)),
        pl.BlockSpec((TOK_BLK // 8, 2, 8, 128), lambda i: (i, 0, 0, 0)),
    ],
    out_shape=[
        jax.ShapeDtypeStruct((NTOK_S // 8, 2, 8, 128), jnp.int32),
        jax.ShapeDtypeStruct((NTOK_S // 8, 2, 8, 128), jnp.float32),
    ],
)


def _sc_body(fi_hbm, w_hbm, val_hbm, out_hbm,
             idx0, idx1, w0, w1, rows0, rows1, ob_v,
             isem0, isem1, gsem0, gsem1):
    wid = lax.axis_index("s") * NC + lax.axis_index("c")
    tok0 = wid * TPW
    idx_b = (idx0, idx1)
    w_b = (w0, w1)
    rows_b = (rows0, rows1)
    isem_b = (isem0, isem1)
    gsem_b = (gsem0, gsem1)

    def fetch_idx(c, b):
        @pl.when(c < NCH)
        def _():
            br = tok0 // CHUNK + c
            pltpu.make_async_copy(fi_hbm.at[br], idx_b[b], isem_b[b]).start()
            pltpu.make_async_copy(w_hbm.at[br], w_b[b], isem_b[b]).start()

    def fire_gathers(c, b):
        @pl.when(c < NCH)
        def _():
            pltpu.make_async_copy(
                fi_hbm.at[0], idx_b[b], isem_b[b]).wait()
            pltpu.make_async_copy(
                w_hbm.at[0], w_b[b], isem_b[b]).wait()
            for t in range(CHUNK):
                for h in range(2):
                    pltpu.make_async_copy(
                        val_hbm.at[idx_b[b].at[h, t]],
                        rows_b[b].at[pl.ds((t * 2 + h) * IDX_PER_DMA,
                                           IDX_PER_DMA)],
                        gsem_b[b]).start()

    def compute(c, b):
        rows_v = rows_b[b]
        w_v = w_b[b]
        for t in range(CHUNK):
            for h in range(2):
                pltpu.make_async_copy(
                    val_hbm.at[idx_b[b].at[h, t]],
                    rows_v.at[pl.ds((t * 2 + h) * IDX_PER_DMA, IDX_PER_DMA)],
                    gsem_b[b]).wait()
        for t in range(CHUNK):
            z = jnp.zeros((VDIM,), jnp.float32)
            accs = (z, z, z, z)
            for h in range(2):
                def row_body(r, a_in, _t=t, _h=h):
                    a = list(a_in)
                    base = r * 32
                    wa = w_v[_h, _t, pl.ds(base, 16)]
                    wb = w_v[_h, _t, pl.ds(base + 16, 16)]
                    rowbase = _t * KK + _h * IDX_PER_DMA + base
                    for u in range(16):
                        a[u % 4] = a[u % 4] + rows_v[rowbase + u, :] * wa[u]
                    for u in range(16):
                        a[u % 4] = (a[u % 4]
                                    + rows_v[rowbase + 16 + u, :] * wb[u])
                    return tuple(a)
                accs = lax.fori_loop(0, IDX_PER_DMA // 32, row_body, accs)
            a0, a1, a2, a3 = accs
            ob_v[t, :] = (a0 + a1) + (a2 + a3)
        pltpu.sync_copy(ob_v, out_hbm.at[pl.ds(tok0 + c * CHUNK, CHUNK)])

    fetch_idx(0, 0)
    fetch_idx(1, 1)
    fire_gathers(0, 0)

    def pair_body(p, carry):
        c0 = 2 * p
        for b in (0, 1):
            c = c0 + b
            fire_gathers(c + 1, 1 - b)
            compute(c, b)
            fetch_idx(c + 2, b)
        return carry

    lax.fori_loop(0, NCH // 2, pair_body, 0)


@functools.cache
def _get_sc_call():
    return pl.kernel(
        _sc_body,
        out_type=jax.ShapeDtypeStruct((NTOK_S, VDIM), jnp.float32),
        mesh=plsc.VectorSubcoreMesh(core_axis_name="c", subcore_axis_name="s",
                                    num_cores=NC, num_subcores=NS),
        compiler_params=pltpu.CompilerParams(use_tc_tiling_on_sc=False),
        scratch_types=[
            pltpu.VMEM((2, CHUNK, IDX_PER_DMA), jnp.int32),
            pltpu.VMEM((2, CHUNK, IDX_PER_DMA), jnp.int32),
            pltpu.VMEM((2, CHUNK, IDX_PER_DMA), jnp.float32),
            pltpu.VMEM((2, CHUNK, IDX_PER_DMA), jnp.float32),
            pltpu.VMEM((ROWS, VDIM), jnp.float32),
            pltpu.VMEM((ROWS, VDIM), jnp.float32),
            pltpu.VMEM((CHUNK, VDIM), jnp.float32),
            pltpu.SemaphoreType.DMA,
            pltpu.SemaphoreType.DMA,
            pltpu.SemaphoreType.DMA,
            pltpu.SemaphoreType.DMA,
        ],
    )


def kernel(query, codebook1, codebook2, values):
    b, t, d = query.shape
    q = query.reshape(b * t, d)
    sc = _get_sc_call()
    outs = []
    for s in range(NSPLIT):
        qs = q[s * NTOK_S:(s + 1) * NTOK_S]
        fi, wc = _tc_call(qs, codebook1, codebook2)
        outs.append(sc(fi, wc, values))
    return jnp.concatenate(outs, 0).reshape(b, t, VDIM)


# speedup vs baseline: 1.2125x; 1.0013x over previous
import functools

import jax
import jax.numpy as jnp
from jax import lax
from jax.experimental import pallas as pl
from jax.experimental.pallas import tpu as pltpu
from jax.experimental.pallas import tpu_sc as plsc

SUB_KEYS = 1024
TOPK = 16
HALF = 512
VDIM = 16
NTOK = 8192
KK = TOPK * TOPK

NVAL = SUB_KEYS * SUB_KEYS
NSPLIT = 2
NTOK_S = NTOK // NSPLIT

TOK_BLK = 256
GRID = NTOK_S // TOK_BLK

NC, NS = 2, 16
NW = NC * NS
TPW = NTOK_S // NW
CHUNK = 8
ROWS = CHUNK * KK
NCH = TPW // CHUNK
IDX_PER_DMA = 128
NDMA = ROWS // IDX_PER_DMA


def _top16_t(s, row):
    vals, idxs = [], []
    for _ in range(TOPK):
        m = jnp.max(s, axis=0, keepdims=True)
        im = jnp.min(jnp.where(s == m, row, SUB_KEYS), axis=0, keepdims=True)
        vals.append(m)
        idxs.append(im)
        s = jnp.where(row == im, -jnp.inf, s)
    return jnp.concatenate(vals, 0), jnp.concatenate(idxs, 0)


def _tc_body(q_ref, c1_ref, c2_ref, fi_ref, wc_ref):
    q = q_ref[...]
    dn = (((1,), (1,)), ((), ()))
    s1 = lax.dot_general(c1_ref[...], q[:, :HALF], dn,
                         preferred_element_type=jnp.float32)
    s2 = lax.dot_general(c2_ref[...], q[:, HALF:], dn,
                         preferred_element_type=jnp.float32)
    row = lax.broadcasted_iota(jnp.int32, (SUB_KEYS, TOK_BLK), 0)
    v1, i1 = _top16_t(s1, row)
    v2, i2 = _top16_t(s2, row)
    e1 = jnp.exp(v1 - v1[0:1])
    w1 = e1 / jnp.sum(e1, axis=0, keepdims=True)
    e2 = jnp.exp(v2 - v2[0:1])
    w2 = e2 / jnp.sum(e2, axis=0, keepdims=True)
    fi_t = (i1[:, None, :] * SUB_KEYS + i2[None, :, :]).reshape(KK, TOK_BLK)
    wc_t = (w1[:, None, :] * w2[None, :, :]).reshape(KK, TOK_BLK)
    fi4 = fi_t.T.reshape(TOK_BLK // 8, 8, 2, 128).transpose(0, 2, 1, 3)
    wc4 = wc_t.T.reshape(TOK_BLK // 8, 8, 2, 128).transpose(0, 2, 1, 3)
    fi_ref[...] = fi4
    wc_ref[...] = wc4


_tc_call = pl.pallas_call(
    _tc_body,
    grid=(GRID,),
    in_specs=[
        pl.BlockSpec((TOK_BLK, 2 * HALF), lambda i: (i, 0)),
        pl.BlockSpec((SUB_KEYS, HALF), lambda i: (0, 0)),
        pl.BlockSpec((SUB_KEYS, HALF), lambda i: (0, 0)),
    ],
    out_specs=[
        pl.BlockSpec((TOK_BLK // 8, 2, 8, 128), lambda i: (i, 0, 0, 0)),
        pl.BlockSpec((TOK_BLK // 8, 2, 8, 128), lambda i: (i, 0, 0, 0)),
    ],
    out_shape=[
        jax.ShapeDtypeStruct((NTOK_S // 8, 2, 8, 128), jnp.int32),
        jax.ShapeDtypeStruct((NTOK_S // 8, 2, 8, 128), jnp.float32),
    ],
)


def _sc_body(fi_hbm, w_hbm, val_hbm, out_hbm,
             idx0, idx1, w0, w1, rows0, rows1, ob_v,
             isem0, isem1, gsem0, gsem1):
    wid = lax.axis_index("s") * NC + lax.axis_index("c")
    tok0 = wid * TPW
    idx_b = (idx0, idx1)
    w_b = (w0, w1)
    rows_b = (rows0, rows1)
    isem_b = (isem0, isem1)
    gsem_b = (gsem0, gsem1)

    def fetch_idx(c, b):
        @pl.when(c < NCH)
        def _():
            br = tok0 // CHUNK + c
            pltpu.make_async_copy(fi_hbm.at[br], idx_b[b], isem_b[b]).start()
            pltpu.make_async_copy(w_hbm.at[br], w_b[b], isem_b[b]).start()

    def fire_gathers(c, b):
        @pl.when(c < NCH)
        def _():
            pltpu.make_async_copy(
                fi_hbm.at[0], idx_b[b], isem_b[b]).wait()
            pltpu.make_async_copy(
                w_hbm.at[0], w_b[b], isem_b[b]).wait()
            for t in range(CHUNK):
                for h in range(2):
                    pltpu.make_async_copy(
                        val_hbm.at[idx_b[b].at[h, t]],
                        rows_b[b].at[pl.ds((t * 2 + h) * IDX_PER_DMA,
                                           IDX_PER_DMA)],
                        gsem_b[b]).start()

    def compute(c, b):
        rows_v = rows_b[b]
        w_v = w_b[b]
        for t in range(CHUNK):
            for h in range(2):
                pltpu.make_async_copy(
                    val_hbm.at[idx_b[b].at[h, t]],
                    rows_v.at[pl.ds((t * 2 + h) * IDX_PER_DMA, IDX_PER_DMA)],
                    gsem_b[b]).wait()
        for t in range(CHUNK):
            z = jnp.zeros((VDIM,), jnp.float32)
            accs = (z, z, z, z)
            for h in range(2):
                def row_body(r, a_in, _t=t, _h=h):
                    a = list(a_in)
                    base = r * 32
                    wa = w_v[_h, _t, pl.ds(base, 16)]
                    wb = w_v[_h, _t, pl.ds(base + 16, 16)]
                    rowbase = _t * KK + _h * IDX_PER_DMA + base
                    for u in range(16):
                        a[u % 4] = a[u % 4] + rows_v[rowbase + u, :] * wa[u]
                    for u in range(16):
                        a[u % 4] = (a[u % 4]
                                    + rows_v[rowbase + 16 + u, :] * wb[u])
                    return tuple(a)
                accs = lax.fori_loop(0, IDX_PER_DMA // 32, row_body, accs)
            a0, a1, a2, a3 = accs
            ob_v[t, :] = (a0 + a1) + (a2 + a3)
        pltpu.sync_copy(ob_v, out_hbm.at[pl.ds(tok0 + c * CHUNK, CHUNK)])

    fetch_idx(0, 0)
    fetch_idx(1, 1)
    fire_gathers(0, 0)

    def pair_body(p, carry):
        c0 = 2 * p
        for b in (0, 1):
            c = c0 + b
            fire_gathers(c + 1, 1 - b)
            compute(c, b)
            fetch_idx(c + 2, b)
        return carry

    lax.fori_loop(0, NCH // 2, pair_body, 0)


@functools.cache
def _get_sc_call():
    return pl.kernel(
        _sc_body,
        out_type=jax.ShapeDtypeStruct((NTOK_S, VDIM), jnp.float32),
        mesh=plsc.VectorSubcoreMesh(core_axis_name="c", subcore_axis_name="s",
                                    num_cores=NC, num_subcores=NS),
        compiler_params=pltpu.CompilerParams(use_tc_tiling_on_sc=False),
        scratch_types=[
            pltpu.VMEM((2, CHUNK, IDX_PER_DMA), jnp.int32),
            pltpu.VMEM((2, CHUNK, IDX_PER_DMA), jnp.int32),
            pltpu.VMEM((2, CHUNK, IDX_PER_DMA), jnp.float32),
            pltpu.VMEM((2, CHUNK, IDX_PER_DMA), jnp.float32),
            pltpu.VMEM((ROWS, VDIM), jnp.float32),
            pltpu.VMEM((ROWS, VDIM), jnp.float32),
            pltpu.VMEM((CHUNK, VDIM), jnp.float32),
            pltpu.SemaphoreType.DMA,
            pltpu.SemaphoreType.DMA,
            pltpu.SemaphoreType.DMA,
            pltpu.SemaphoreType.DMA,
        ],
    )


def kernel(query, codebook1, codebook2, values):
    b, t, d = query.shape
    q = query.reshape(b * t, d)
    vwide = lax.optimization_barrier(values.reshape(NVAL // 8, 128))
    vlin = vwide.reshape(NVAL, VDIM)
    sc = _get_sc_call()
    outs = []
    for s in range(NSPLIT):
        qs = q[s * NTOK_S:(s + 1) * NTOK_S]
        fi, wc = _tc_call(qs, codebook1, codebook2)
        outs.append(sc(fi, wc, vlin))
    return jnp.concatenate(outs, 0).reshape(b, t, VDIM)
